# Initial kernel scaffold; baseline (speedup 1.0000x reference)
import jax
import jax.numpy as jnp
from jax.experimental import pallas as pl


def kernel(inputs, table):
    raise NotImplementedError("write your pallas kernel here")


# speedup vs baseline: 5.9921x; 5.9921x over previous
import functools

import jax
import jax.numpy as jnp
from jax import lax
from jax.experimental import pallas as pl
from jax.experimental.pallas import tpu as pltpu
from jax.experimental.pallas import tpu_sc as plsc


@functools.lru_cache(maxsize=None)
def _build(B, D, chunk):
    info = plsc.get_sparse_core_info()
    nw = info.num_cores * info.num_subcores
    b_per_w = B // nw
    n_chunks = b_per_w // chunk
    assert b_per_w % chunk == 0 and chunk % 8 == 0

    mesh = plsc.VectorSubcoreMesh(core_axis_name="c", subcore_axis_name="s")

    @functools.partial(
        pl.kernel,
        mesh=mesh,
        out_type=jax.ShapeDtypeStruct((B, D), jnp.float32),
        scratch_types=[
            pltpu.VMEM((chunk,), jnp.int32),
            pltpu.VMEM((chunk, D), jnp.float32),
            pltpu.SemaphoreType.DMA,
        ],
        compiler_params=pltpu.CompilerParams(use_tc_tiling_on_sc=False),
    )
    def gather_kernel(idx_hbm, table_hbm, out_hbm, idx_v, rows_v, sem):
        wid = lax.axis_index("s") * info.num_cores + lax.axis_index("c")
        base = wid * b_per_w

        def body(i, carry):
            off = base + i * chunk
            pltpu.sync_copy(idx_hbm.at[pl.ds(off, chunk)], idx_v)
            pltpu.async_copy(table_hbm.at[idx_v], rows_v, sem).wait()
            pltpu.sync_copy(rows_v, out_hbm.at[pl.ds(off, chunk)])
            return carry

        lax.fori_loop(0, n_chunks, body, 0)

    return gather_kernel


def kernel(inputs, table):
    batch, hist = inputs.shape
    vocab, dim = table.shape
    flat_idx = inputs.reshape(-1).astype(jnp.int32)
    out = _build(batch * hist, dim, 800)(flat_idx, table)
    return out.reshape(batch, hist, dim)


# speedup vs baseline: 6.2217x; 1.0383x over previous
import functools

import jax
import jax.numpy as jnp
from jax import lax
from jax.experimental import pallas as pl
from jax.experimental.pallas import tpu as pltpu
from jax.experimental.pallas import tpu_sc as plsc


@functools.lru_cache(maxsize=None)
def _build(B, D, chunk):
    info = plsc.get_sparse_core_info()
    nw = info.num_cores * info.num_subcores
    b_per_w = B // nw
    n_chunks = b_per_w // chunk
    n2 = n_chunks // 2
    assert B % nw == 0 and b_per_w % chunk == 0 and chunk % 8 == 0
    assert n_chunks % 2 == 0 and n2 >= 3

    mesh = plsc.VectorSubcoreMesh(core_axis_name="c", subcore_axis_name="s")

    @functools.partial(
        pl.kernel,
        mesh=mesh,
        out_type=jax.ShapeDtypeStruct((B, D), jnp.float32),
        scratch_types=[
            pltpu.VMEM((n_chunks, chunk), jnp.int32),
            pltpu.VMEM((chunk, D), jnp.float32),
            pltpu.VMEM((chunk, D), jnp.float32),
            pltpu.SemaphoreType.DMA,
            pltpu.SemaphoreType.DMA,
            pltpu.SemaphoreType.DMA,
            pltpu.SemaphoreType.DMA,
        ],
        compiler_params=pltpu.CompilerParams(use_tc_tiling_on_sc=False),
    )
    def gather_kernel(idx_hbm, table_hbm, out_hbm, idx_all, rows0, rows1,
                      g0, g1, o0, o1):
        wid = lax.axis_index("s") * info.num_cores + lax.axis_index("c")
        base = wid * b_per_w

        pltpu.sync_copy(idx_hbm.at[wid], idx_all)

        def gat(i, rows, sem):
            return pltpu.make_async_copy(table_hbm.at[idx_all.at[i]], rows, sem)

        def st(i, rows, sem):
            return pltpu.make_async_copy(
                rows, out_hbm.at[pl.ds(base + i * chunk, chunk)], sem)

        gat(0, rows0, g0).start()
        gat(0, rows0, g0).wait()
        st(0, rows0, o0).start()
        gat(1, rows1, g1).start()
        gat(1, rows1, g1).wait()
        st(1, rows1, o1).start()
        st(0, rows0, o0).wait()
        gat(2, rows0, g0).start()

        def body(g, carry):
            i0 = 2 * g
            i1 = i0 + 1
            st(i0 - 1, rows1, o1).wait()
            gat(i0, rows0, g0).wait()
            st(i0, rows0, o0).start()
            gat(i1, rows1, g1).start()
            gat(i1, rows1, g1).wait()
            st(i1, rows1, o1).start()
            st(i0, rows0, o0).wait()
            gat(i0 + 2, rows0, g0).start()
            return carry

        lax.fori_loop(1, n2 - 1, body, 0)

        i0 = n_chunks - 2
        i1 = n_chunks - 1
        st(i0 - 1, rows1, o1).wait()
        gat(i0, rows0, g0).wait()
        st(i0, rows0, o0).start()
        gat(i1, rows1, g1).start()
        gat(i1, rows1, g1).wait()
        st(i1, rows1, o1).start()
        st(i0, rows0, o0).wait()
        st(i1, rows1, o1).wait()

    return gather_kernel


def kernel(inputs, table):
    batch, hist = inputs.shape
    vocab, dim = table.shape
    B = batch * hist
    chunk = 800
    info = plsc.get_sparse_core_info()
    nw = info.num_cores * info.num_subcores
    n_chunks = B // nw // chunk
    idx = inputs.reshape(nw, n_chunks, chunk).astype(jnp.int32)
    out = _build(B, dim, chunk)(idx, table)
    return out.reshape(batch, hist, dim)
